# baseline (speedup 1.0000x reference)
import functools

import jax
import jax.numpy as jnp
from jax import lax
from jax.experimental import pallas as pl
from jax.experimental.pallas import tpu as pltpu
from jax.experimental.pallas import tpu_sc as plsc

VOCAB = 1000000
EMBED_DIM = 32
BATCH = 4096
HIST = 50

_B = BATCH * HIST

_info = plsc.get_sparse_core_info()
_NC = _info.num_cores
_NS = _info.num_subcores
_NW = _NC * _NS
_B_PER_W = _B // _NW
_NBLK = _B_PER_W // 128
_NPAIR = _NBLK // 2

_mesh = plsc.VectorSubcoreMesh(core_axis_name="c", subcore_axis_name="s")


@functools.partial(
    pl.kernel,
    mesh=_mesh,
    out_type=jax.ShapeDtypeStruct((HIST * 4 * 32, 8, 128), jnp.float32),
    scratch_types=[
        pltpu.VMEM((_B_PER_W,), jnp.int32),
        pltpu.VMEM((2, 4, 8, 128), jnp.float32),
        pltpu.SemaphoreType.DMA,
        pltpu.SemaphoreType.DMA,
        pltpu.SemaphoreType.DMA,
        pltpu.SemaphoreType.DMA,
    ],
    compiler_params=pltpu.CompilerParams(
        use_tc_tiling_on_sc=False, needs_layout_passes=False
    ),
)
def _sc_gather(table_hbm, idx_hbm, out_hbm, idx_v, t_buf, g0, g1, w0, w1):
    wid = lax.axis_index("s") * _NC + lax.axis_index("c")
    base = wid * _B_PER_W
    pltpu.sync_copy(idx_hbm.at[pl.ds(base, _B_PER_W)], idx_v)

    def gather_args(blk, buf, c, sem):
        return (
            table_hbm.at[c].at[idx_v.at[pl.ds(blk * 128, 128)]],
            t_buf.at[buf, c // 8, c % 8],
            sem,
        )

    def wb_args(blk, buf, c4, sem):
        blk_g = wid * _NBLK + blk
        h = lax.shift_right_logical(blk_g, 5)
        bt = lax.bitwise_and(blk_g, 31)
        return (
            t_buf.at[buf, c4],
            out_hbm.at[(h * 4 + c4) * 32 + bt],
            sem,
        )

    def issue_g(blk, buf, sem):
        for c in range(EMBED_DIM):
            pltpu.async_copy(*gather_args(blk, buf, c, sem))

    def drain_g(blk, buf, sem):
        for c in range(EMBED_DIM):
            pltpu.make_async_copy(*gather_args(blk, buf, c, sem)).wait()

    def issue_w(blk, buf, sem):
        for c4 in range(4):
            pltpu.async_copy(*wb_args(blk, buf, c4, sem))

    def drain_w(blk, buf, sem):
        for c4 in range(4):
            pltpu.make_async_copy(*wb_args(blk, buf, c4, sem)).wait()

    def pair(k, carry):
        a = 2 * k
        b = a + 1
        drain_g(a, 0, g0)
        issue_w(a, 0, w0)
        drain_g(b, 1, g1)
        issue_w(b, 1, w1)
        drain_w(a, 0, w0)
        issue_g(a + 2, 0, g0)
        drain_w(b, 1, w1)
        issue_g(b + 2, 1, g1)
        return carry

    issue_g(0, 0, g0)
    issue_g(1, 1, g1)
    lax.fori_loop(0, _NPAIR - 1, pair, 0)
    last = 2 * (_NPAIR - 1)
    drain_g(last, 0, g0)
    issue_w(last, 0, w0)
    drain_g(last + 1, 1, g1)
    issue_w(last + 1, 1, w1)
    drain_w(last, 0, w0)
    drain_w(last + 1, 1, w1)


def kernel(inputs, embedding):
    ids = jnp.transpose(inputs).reshape(_B).astype(jnp.int32)
    emb_t = jnp.transpose(embedding)
    v = _sc_gather(emb_t, ids)
    return (
        v.reshape(HIST, 4, 32, 8, 128)
        .transpose(2, 4, 0, 1, 3)
        .reshape(BATCH, HIST, EMBED_DIM)
    )


# speedup vs baseline: 4.3697x; 4.3697x over previous
import functools

import jax
import jax.numpy as jnp
from jax import lax
from jax.experimental import pallas as pl
from jax.experimental.pallas import tpu as pltpu
from jax.experimental.pallas import tpu_sc as plsc

VOCAB = 1000000
EMBED_DIM = 32
BATCH = 4096
HIST = 50

_B = BATCH * HIST

_info = plsc.get_sparse_core_info()
_NC = _info.num_cores
_NS = _info.num_subcores
_NW = _NC * _NS
_B_PER_W = _B // _NW
_NBLK = _B_PER_W // 128
_NB = 5
_NCH = _NBLK // _NB
_CH_IDS = _NB * 128
_TWORDS = _NB * 4 * 8 * 128

_mesh = plsc.VectorSubcoreMesh(core_axis_name="c", subcore_axis_name="s")


@functools.partial(
    pl.kernel,
    mesh=_mesh,
    out_type=jax.ShapeDtypeStruct((HIST, 4, 32, 1024), jnp.float32),
    scratch_types=[
        pltpu.VMEM((_B_PER_W,), jnp.int32),
        pltpu.VMEM((2, _CH_IDS, EMBED_DIM), jnp.float32),
        pltpu.VMEM((2, _TWORDS), jnp.float32),
        pltpu.SemaphoreType.DMA,
        pltpu.SemaphoreType.DMA,
        pltpu.SemaphoreType.DMA,
        pltpu.SemaphoreType.DMA,
    ],
    compiler_params=pltpu.CompilerParams(
        use_tc_tiling_on_sc=False, needs_layout_passes=False
    ),
)
def _sc_gather(table_hbm, idx_hbm, out_hbm, idx_v, g_buf, t_buf, g0, g1, w0, w1):
    wid = lax.axis_index("s") * _NC + lax.axis_index("c")
    base = wid * _B_PER_W
    gsem = (g0, g1)
    wsem = (w0, w1)
    pltpu.sync_copy(idx_hbm.at[pl.ds(base, _B_PER_W)], idx_v)

    def gather_args(ch):
        return (
            table_hbm.at[idx_v.at[pl.ds(ch * _CH_IDS, _CH_IDS)]],
            g_buf.at[ch % 2],
            gsem[ch % 2],
        )

    def wb_args(ch, lb, c4):
        blk = wid * _NBLK + ch * _NB + lb
        h = lax.shift_right_logical(blk, 5)
        bt = lax.bitwise_and(blk, 31)
        return (
            t_buf.at[ch % 2, pl.ds(c4 * (_NB * 1024) + lb * 1024, 1024)],
            out_hbm.at[h, c4, bt],
            wsem[ch % 2],
        )

    lane = lax.iota(jnp.int32, 16)
    pat = (lane // 8) * (_NB * 1024) + (lane % 8) * 128

    def transpose(ch):
        gb = g_buf.at[ch % 2]
        tb = t_buf.at[ch % 2]

        for lb in range(_NB):
            def body(j, dst, lb=lb):
                m = lb * 128 + j
                val0 = gb[m, pl.ds(0, 16)]
                plsc.store_scatter(tb, [dst], val0)
                val1 = gb[m, pl.ds(16, 16)]
                plsc.store_scatter(tb, [dst + 2 * _NB * 1024], val1)
                return dst + 1

            lax.fori_loop(0, 128, body, pat + lb * 1024, unroll=8)

    def issue_wb(ch):
        for lb in range(_NB):
            for c4 in range(4):
                pltpu.async_copy(*wb_args(ch, lb, c4))

    def drain_wb(ch):
        for lb in range(_NB):
            for c4 in range(4):
                pltpu.make_async_copy(*wb_args(ch, lb, c4)).wait()

    pltpu.async_copy(*gather_args(0))
    for ch in range(_NCH):
        pltpu.make_async_copy(*gather_args(ch)).wait()
        if ch >= 1:
            issue_wb(ch - 1)
        if ch + 1 < _NCH:
            pltpu.async_copy(*gather_args(ch + 1))
        if ch >= 2:
            drain_wb(ch - 2)
        transpose(ch)
    drain_wb(_NCH - 2)
    issue_wb(_NCH - 1)
    drain_wb(_NCH - 1)


def kernel(inputs, embedding):
    ids = jnp.transpose(inputs).reshape(_B).astype(jnp.int32)
    v = _sc_gather(embedding, ids)
    return (
        v.reshape(HIST, 4, 32, 8, 128)
        .transpose(2, 4, 0, 1, 3)
        .reshape(BATCH, HIST, EMBED_DIM)
    )
